# Initial kernel scaffold; baseline (speedup 1.0000x reference)
import jax
import jax.numpy as jnp
from jax.experimental import pallas as pl


def kernel(coordinates, directions, points_position, tex_tplanes, pq_emb, pq_w1, pq_b1, pq_w2, pq_b2, f_w1, f_b1, f_w2, f_b2, f_w3, f_b3, d_w, d_b, r_w1, r_b1, r_w2, r_b2):
    raise NotImplementedError("write your pallas kernel here")


# speedup vs baseline: 20.1893x; 20.1893x over previous
import functools

import jax
import jax.numpy as jnp
import numpy as np
from jax import lax
from jax.experimental import pallas as pl
from jax.experimental.pallas import tpu as pltpu
from jax.experimental.pallas import tpu_sc as plsc

N, M, P, CH, HW, K, OUT = 2, 2048, 8192, 32, 256, 4, 3
NM = N * M
BM = 256
MB = M // BM
BMC = 512
NPLANES = 3
NCORNERS = 4 * NPLANES
DPT = 128

def _harm_perm(base):
    sin = [base + c * 4 + f for f in range(4) for c in range(3)]
    cos = [base + 12 + c * 4 + f for f in range(4) for c in range(3)]
    raw = [base + 24 + c for c in range(3)]
    return sin + cos + raw

_PERM59 = np.array(list(range(32)) + _harm_perm(32), dtype=np.int32)
_PERM155 = np.array(list(range(128)) + _harm_perm(128), dtype=np.int32)


def _stage_a_body(c8_ref, ptst_ref, dist_ref, idxp_ref, idxt_ref, cw_ref,
                  inv_ref):
    n = pl.program_id(0)
    mb = pl.program_id(1)
    c8 = c8_ref[...]
    ptst = ptst_ref[0]
    csq = jnp.sum(c8 * c8, axis=1, keepdims=True)
    psq = jnp.sum(ptst * ptst, axis=0, keepdims=True)
    dot = lax.dot_general(c8, ptst, (((1,), (0,)), ((), ())),
                          preferred_element_type=jnp.float32)
    d2 = csq + psq - 2.0 * dot
    iota = lax.broadcasted_iota(jnp.int32, (BM, P), 1)

    dist_cols, idx_cols, inv_rows = [], [], []
    for _ in range(K):
        mval = jnp.min(d2, axis=1, keepdims=True)
        idxk = jnp.min(jnp.where(d2 == mval, iota, P), axis=1,
                       keepdims=True)
        d2 = jnp.where(iota == idxk, jnp.float32(jnp.inf), d2)
        dist_cols.append(mval)
        idx_cols.append(idxk)
        inv_rows.append(jnp.full((1, 128), jnp.sum(1.0 / mval)))
    zf = jnp.zeros((BM, 8 - K), jnp.float32)
    zi = jnp.zeros((BM, 8 - K), jnp.int32)
    dist_ref[...] = jnp.concatenate(dist_cols + [zf], axis=1)
    idxp_ref[...] = jnp.concatenate(idx_cols + [zi], axis=1) + n * P

    inv_new = jnp.concatenate(
        inv_rows + [jnp.zeros((8 - K, 128), jnp.float32)], axis=0)[None]

    @pl.when(mb == 0)
    def _():
        inv_ref[...] = inv_new

    @pl.when(mb > 0)
    def _():
        inv_ref[...] = inv_ref[...] + inv_new

    ci_cols, cwt_cols = [], []
    for p_i, (ux, vy) in enumerate(((0, 1), (0, 2), (2, 1))):
        x = (c8[:, ux:ux + 1] + 1.0) * (0.5 * HW) - 0.5
        y = (c8[:, vy:vy + 1] + 1.0) * (0.5 * HW) - 0.5
        x0 = jnp.floor(x)
        y0 = jnp.floor(y)
        wx1 = x - x0
        wy1 = y - y0
        base = (n * NPLANES + p_i) * (HW * HW)
        for dx, dy, wgt in ((0, 0, (1.0 - wx1) * (1.0 - wy1)),
                            (1, 0, wx1 * (1.0 - wy1)),
                            (0, 1, (1.0 - wx1) * wy1),
                            (1, 1, wx1 * wy1)):
            xi = x0 + dx
            yi = y0 + dy
            valid = ((xi >= 0) & (xi < HW) & (yi >= 0) & (yi < HW))
            xc = jnp.clip(xi, 0, HW - 1).astype(jnp.int32)
            yc = jnp.clip(yi, 0, HW - 1).astype(jnp.int32)
            ci_cols.append(base + yc * HW + xc)
            cwt_cols.append(wgt * valid.astype(jnp.float32) * (1.0 / 3.0))
    zi4 = jnp.zeros((BM, 16 - NCORNERS), jnp.int32)
    zf4 = jnp.zeros((BM, 16 - NCORNERS), jnp.float32)
    idxt_ref[...] = jnp.concatenate(ci_cols + [zi4], axis=1)
    cw_ref[...] = jnp.concatenate(cwt_cols + [zf4], axis=1)


def _stage_a(c8, ptst8):
    return pl.pallas_call(
        _stage_a_body,
        grid=(N, MB),
        in_specs=[
            pl.BlockSpec((BM, 8), lambda n, mb: (n * MB + mb, 0)),
            pl.BlockSpec((1, 8, P), lambda n, mb: (n, 0, 0)),
        ],
        out_specs=[
            pl.BlockSpec((BM, 8), lambda n, mb: (n * MB + mb, 0)),
            pl.BlockSpec((BM, 8), lambda n, mb: (n * MB + mb, 0)),
            pl.BlockSpec((BM, 16), lambda n, mb: (n * MB + mb, 0)),
            pl.BlockSpec((BM, 16), lambda n, mb: (n * MB + mb, 0)),
            pl.BlockSpec((1, 8, 128), lambda n, mb: (n, 0, 0)),
        ],
        out_shape=[
            jax.ShapeDtypeStruct((NM, 8), jnp.float32),
            jax.ShapeDtypeStruct((NM, 8), jnp.int32),
            jax.ShapeDtypeStruct((NM, 16), jnp.int32),
            jax.ShapeDtypeStruct((NM, 16), jnp.float32),
            jax.ShapeDtypeStruct((N, 8, 128), jnp.float32),
        ],
    )(c8, ptst8)


def _sc_gather(table_pts, table_tex, idxp_t, idxt_t):
    info = plsc.get_sparse_core_info()
    nw = info.num_cores * info.num_subcores
    chunk = NM // nw
    mesh = plsc.VectorSubcoreMesh(core_axis_name="c", subcore_axis_name="s")

    @functools.partial(
        pl.kernel, mesh=mesh,
        out_type=[
            jax.ShapeDtypeStruct((K, NM, DPT), jnp.float32),
            jax.ShapeDtypeStruct((NCORNERS, NM, DPT), jnp.float32),
        ],
        scratch_types=[
            pltpu.VMEM((K, chunk), jnp.int32),
            pltpu.VMEM((NCORNERS, chunk), jnp.int32),
            pltpu.VMEM((4, chunk, DPT), jnp.float32),
            pltpu.SemaphoreType.DMA,
        ],
    )
    def run(tp_hbm, tt_hbm, ip_hbm, it_hbm, pf_hbm, to_hbm,
            ipv, itv, buf, sem):
        wid = lax.axis_index("s") * info.num_cores + lax.axis_index("c")
        base = wid * chunk
        for s in range(K):
            pltpu.sync_copy(ip_hbm.at[s, pl.ds(base, chunk)], ipv.at[s])
        for s in range(NCORNERS):
            pltpu.sync_copy(it_hbm.at[s, pl.ds(base, chunk)], itv.at[s])
        groups = [(tp_hbm, ipv, pf_hbm, (0, 1, 2, 3)),
                  (tt_hbm, itv, to_hbm, (0, 1, 2, 3)),
                  (tt_hbm, itv, to_hbm, (4, 5, 6, 7)),
                  (tt_hbm, itv, to_hbm, (8, 9, 10, 11))]
        for table, idxv, outh, rows in groups:
            cps = [pltpu.async_copy(table.at[idxv.at[r]], buf.at[j], sem)
                   for j, r in enumerate(rows)]
            for cp in cps:
                cp.wait()
            for j, r in enumerate(rows):
                pltpu.sync_copy(buf.at[j], outh.at[r, pl.ds(base, chunk)])

    return run(table_pts, table_tex, idxp_t, idxt_t)


def _lin(x, w, b):
    return lax.dot_general(x, w, (((1,), (1,)), ((), ())),
                           preferred_element_type=jnp.float32) + b


def _sincos_fm(v):
    xs = jnp.concatenate([v, 2.0 * v, 4.0 * v, 8.0 * v], axis=1)
    return jnp.sin(xs), jnp.cos(xs)


def _stage_c_body(c8_ref, d8_ref, dist_ref, cw_ref, inv_ref, pf_ref, tex_ref,
                  w1_ref, b1_ref, w2_ref, b2_ref,
                  fw1_ref, fb1_ref, fw2_ref, fb2_ref, fw3_ref, fb3_ref,
                  dw_ref, db_ref, rw1_ref, rb1_ref, rw2_ref, rb2_ref,
                  dens_ref, rgb_ref):
    c = c8_ref[...][:, :3]
    inside = (c > -1.0) & (c < 1.0)
    sel = jnp.min(jnp.where(inside, 1.0, 0.0), axis=1, keepdims=True)

    cwv = cw_ref[...]
    tex = cwv[:, 0:1] * tex_ref[0][:, :CH]
    for j in range(1, NCORNERS):
        tex = tex + cwv[:, j:j + 1] * tex_ref[j][:, :CH]

    dist = dist_ref[...]
    sp = jnp.zeros((BMC, 32), jnp.float32)
    for k in range(K):
        row = pf_ref[k]
        emb = row[:, :32]
        pos = row[:, 32:35]
        rel = c - pos
        nrm = jnp.sqrt(jnp.sum(rel * rel, axis=1, keepdims=True))
        rel = rel / jnp.maximum(nrm, 1e-12)
        sn, cs = _sincos_fm(rel)
        pin = jnp.concatenate([emb, sn, cs, rel], axis=1)
        h = jnp.maximum(_lin(pin, w1_ref[...], b1_ref[...]), 0.0)
        pq = _lin(h, w2_ref[...], b2_ref[...])
        wk = (1.0 / dist[:, k:k + 1]) / inv_ref[0, k:k + 1, :1]
        sp = sp + pq * wk

    feat0 = jnp.concatenate([tex, sp], axis=1)
    f1 = jnp.maximum(_lin(feat0, fw1_ref[...], fb1_ref[...]), 0.0)
    f2 = jnp.maximum(_lin(f1, fw2_ref[...], fb2_ref[...]), 0.0)
    feat = _lin(f2, fw3_ref[...], fb3_ref[...])

    raw = jax.nn.softplus(10.0 * _lin(feat, dw_ref[...], db_ref[...])) / 10.0
    dens_ref[...] = 1.0 - jnp.exp(-(raw * sel))

    d = d8_ref[...][:, :3]
    dn = d / jnp.maximum(
        jnp.sqrt(jnp.sum(d * d, axis=1, keepdims=True)), 1e-12)
    sn, cs = _sincos_fm(dn)
    rin = jnp.concatenate([feat, sn, cs, dn], axis=1)
    r1 = jnp.maximum(_lin(rin, rw1_ref[...], rb1_ref[...]), 0.0)
    rgb = _lin(r1, rw2_ref[...], rb2_ref[...])
    rgb_ref[...] = jax.nn.sigmoid(rgb) * (1.0 + 2 * 0.001) - 0.001


def _stage_c(c8, d8, dist8, cw16, invsum, pf_rows, tex_rows, weights):
    nblk = NM // BMC

    def _row(i):
        return (i, 0)

    wspecs = [pl.BlockSpec(w.shape, lambda i, nd=w.ndim: (0,) * nd)
              for w in weights]
    return pl.pallas_call(
        _stage_c_body,
        grid=(nblk,),
        in_specs=[
            pl.BlockSpec((BMC, 8), _row),
            pl.BlockSpec((BMC, 8), _row),
            pl.BlockSpec((BMC, 8), _row),
            pl.BlockSpec((BMC, 16), _row),
            pl.BlockSpec((1, 8, 128), lambda i: (i * BMC // M, 0, 0)),
            pl.BlockSpec((K, BMC, DPT), lambda i: (0, i, 0)),
            pl.BlockSpec((NCORNERS, BMC, DPT), lambda i: (0, i, 0)),
        ] + wspecs,
        out_specs=[
            pl.BlockSpec((BMC, 8), _row),
            pl.BlockSpec((BMC, 8), _row),
        ],
        out_shape=[
            jax.ShapeDtypeStruct((NM, 8), jnp.float32),
            jax.ShapeDtypeStruct((NM, 8), jnp.float32),
        ],
    )(c8, d8, dist8, cw16, invsum, pf_rows, tex_rows, *weights)


def kernel(coordinates, directions, points_position, tex_tplanes, pq_emb,
           pq_w1, pq_b1, pq_w2, pq_b2, f_w1, f_b1, f_w2, f_b2, f_w3, f_b3,
           d_w, d_b, r_w1, r_b1, r_w2, r_b2):
    c8 = jnp.pad(coordinates.reshape(NM, 3), ((0, 0), (0, 5)))
    d8 = jnp.pad(directions.reshape(NM, 3), ((0, 0), (0, 5)))
    ptst8 = jnp.pad(points_position.transpose(0, 2, 1), ((0, 0), (0, 5), (0, 0)))

    dist8, idxp8, idxt16, cw16, invsum = _stage_a(c8, ptst8)

    table_pts = jnp.concatenate(
        [jnp.broadcast_to(pq_emb[None], (N, P, 32)), points_position,
         jnp.zeros((N, P, DPT - 35), jnp.float32)], axis=2).reshape(N * P, DPT)
    table_tex = jnp.pad(
        tex_tplanes.transpose(0, 1, 3, 4, 2),
        ((0, 0), (0, 0), (0, 0), (0, 0), (0, DPT - CH))).reshape(
            N * NPLANES * HW * HW, DPT)
    idxp_t = idxp8.T[:K]
    idxt_t = idxt16.T[:NCORNERS]

    pf_rows, tex_rows = _sc_gather(table_pts, table_tex, idxp_t, idxt_t)

    weights = (
        pq_w1[:, _PERM59], pq_b1.reshape(1, 64),
        pq_w2, pq_b2.reshape(1, 32),
        f_w1, f_b1.reshape(1, 128),
        f_w2, f_b2.reshape(1, 128),
        f_w3, f_b3.reshape(1, 128),
        jnp.pad(d_w, ((0, 7), (0, 0))), jnp.pad(d_b.reshape(1, 1), ((0, 0), (0, 7))),
        r_w1[:, _PERM155], r_b1.reshape(1, 64),
        jnp.pad(r_w2, ((0, 8 - OUT), (0, 0))),
        jnp.pad(r_b2.reshape(1, OUT), ((0, 0), (0, 8 - OUT))),
    )
    dens8, rgb8 = _stage_c(c8, d8, dist8, cw16, invsum, pf_rows, tex_rows,
                           weights)

    densities = dens8[:, :1].reshape(N, M, 1)
    rgb = rgb8[:, :OUT].reshape(N, M, OUT)
    dist = dist8[:, :K].reshape(N, M, K)
    return densities, rgb, dist
